# baseline (speedup 1.0000x reference)
import jax
import jax.numpy as jnp
from jax.experimental import pallas as pl
from jax.experimental.pallas import tpu as pltpu

_LIMIT = float(jnp.pi) / 2.0
_WEIGHT = 0.01
_COLS = (9, 10, 24)

_N = 16384
_D = 72
_LANES = 128
_ROWS = _N * _D // _LANES
_BLK = 1152
_GRID = _ROWS // _BLK


def _tc_body(x_ref, o_ref, acc_ref):
    i = pl.program_id(0)

    @pl.when(i == 0)
    def _():
        acc_ref[0] = 0.0

    x = x_ref[...]
    r = jax.lax.broadcasted_iota(jnp.int32, x.shape, 0)
    l = jax.lax.broadcasted_iota(jnp.int32, x.shape, 1)
    col = (r * _LANES + l) % _D
    keep = (col == _COLS[0]) | (col == _COLS[1]) | (col == _COLS[2])
    keep = keep & (jnp.abs(x) > _LIMIT)
    p = jnp.where(keep, x, 0.0)
    acc_ref[0] += jnp.sum(p * p)

    @pl.when(i == _GRID - 1)
    def _():
        o_ref[0] = acc_ref[0] * _WEIGHT


@jax.jit
def kernel(pose):
    flat = pose.reshape(_ROWS, _LANES)
    out = pl.pallas_call(
        _tc_body,
        grid=(_GRID,),
        in_specs=[pl.BlockSpec((_BLK, _LANES), lambda i: (i, 0))],
        out_specs=pl.BlockSpec(memory_space=pltpu.SMEM),
        out_shape=jax.ShapeDtypeStruct((1,), jnp.float32),
        scratch_shapes=[pltpu.SMEM((1,), jnp.float32)],
        compiler_params=pltpu.CompilerParams(
            dimension_semantics=("arbitrary",),
        ),
    )(flat)
    return out[0]


# speedup vs baseline: 3.0861x; 3.0861x over previous
import jax
import jax.numpy as jnp
from jax.experimental import pallas as pl
from jax.experimental.pallas import tpu as pltpu

_LIMIT = float(jnp.pi) / 2.0
_WEIGHT = 0.01
_COLS = (9, 10, 24)

_N = 16384
_D = 72


def _tc_body(a_ref, b_ref, c_ref, o_ref):
    acc = jnp.float32(0.0)
    for ref in (a_ref, b_ref, c_ref):
        v = ref[...]
        p = jnp.where(jnp.abs(v) > _LIMIT, v, 0.0)
        acc = acc + jnp.sum(p * p)
    o_ref[0] = acc * _WEIGHT


@jax.jit
def kernel(pose):
    xt = pose.T.reshape(_D * _N // 128, 128)
    out = pl.pallas_call(
        _tc_body,
        grid=(1,),
        in_specs=[
            pl.BlockSpec((128, 128), lambda i, c=c: (c, 0)) for c in _COLS
        ],
        out_specs=pl.BlockSpec(memory_space=pltpu.SMEM),
        out_shape=jax.ShapeDtypeStruct((1,), jnp.float32),
    )(xt, xt, xt)
    return out[0]


# speedup vs baseline: 12.3042x; 3.9870x over previous
import jax
import jax.numpy as jnp
from jax.experimental import pallas as pl
from jax.experimental.pallas import tpu as pltpu

_LIMIT = float(jnp.pi) / 2.0
_WEIGHT = 0.01

_N = 16384
_D = 72
_BANDS = (1, 3)
_BAND_ROWS = ((1, 2), (0,))


def _tc_body(a_ref, b_ref, o_ref):
    acc = jnp.float32(0.0)
    for ref, rows in zip((a_ref, b_ref), _BAND_ROWS):
        v = ref[...]
        r = jax.lax.broadcasted_iota(jnp.int32, v.shape, 0)
        keep = r == rows[0]
        for extra in rows[1:]:
            keep = keep | (r == extra)
        keep = keep & (jnp.abs(v) > _LIMIT)
        p = jnp.where(keep, v, 0.0)
        acc = acc + jnp.sum(p * p)
    o_ref[0] = acc * _WEIGHT


@jax.jit
def kernel(pose):
    xt = pose.T
    out = pl.pallas_call(
        _tc_body,
        grid=(1,),
        in_specs=[
            pl.BlockSpec((8, _N), lambda i, b=b: (b, 0)) for b in _BANDS
        ],
        out_specs=pl.BlockSpec(memory_space=pltpu.SMEM),
        out_shape=jax.ShapeDtypeStruct((1,), jnp.float32),
    )(xt, xt)
    return out[0]
